# Initial kernel scaffold; baseline (speedup 1.0000x reference)
import functools

import jax
import jax.numpy as jnp
from jax import lax
from jax.experimental import pallas as pl
from jax.experimental.pallas import tpu as pltpu
from jax.experimental.pallas import tpu_sc as plsc

N = 10000
E = 320000
D = 128
L = 16
NC = 2
NS = 16
NW = NC * NS
EW = E // NW
C = 80
NCHUNK = EW // C

ZROWS = 80
FULL_ZCHUNKS = 8
LAST_ZCHUNKS = 5


def _sc_body(x_hbm, src_hbm, dst_hbm, ew_hbm, wmp_hbm, partial_hbm,
             src_v, dst_v, w_v, rows_v, wmp_v, acc, sem):
    cid = lax.axis_index("c")
    sid = lax.axis_index("s")
    wid = cid * NS + sid

    zero = jnp.zeros((L,), jnp.float32)
    def zfill(i, _):
        for j in range(D // L):
            rows_v[i, pl.ds(j * L, L)] = zero
        return 0
    lax.fori_loop(0, C, zfill, 0)
    nz = jnp.where(sid == NS - 1, LAST_ZCHUNKS, FULL_ZCHUNKS)
    zbase = sid * (FULL_ZCHUNKS * ZROWS)
    def zcopy(k, _):
        pltpu.sync_copy(rows_v, acc.at[pl.ds(zbase + k * ZROWS, ZROWS)])
        return 0
    lax.fori_loop(0, nz, zcopy, 0)
    plsc.subcore_barrier()

    tbase = wid * NCHUNK
    pltpu.sync_copy(src_hbm.at[pl.ds(tbase, NCHUNK)], src_v)
    pltpu.sync_copy(dst_hbm.at[pl.ds(tbase, NCHUNK)], dst_v)
    pltpu.sync_copy(ew_hbm.at[pl.ds(tbase, NCHUNK)], w_v)
    pltpu.sync_copy(wmp_hbm, wmp_v)
    wmp = wmp_v[...]

    def chunk(c, _):
        pltpu.async_copy(x_hbm.at[src_v.at[c]], rows_v, sem).wait()
        def row(i, _):
            wb = plsc.load_gather(
                w_v, [jnp.full((L,), c, jnp.int32), jnp.full((L,), i, jnp.int32)])
            wb = wb * wmp
            for j in range(D // L):
                rows_v[i, pl.ds(j * L, L)] = rows_v[i, pl.ds(j * L, L)] * wb
            return 0
        lax.fori_loop(0, C, row, 0)
        pltpu.sync_copy(rows_v, acc.at[dst_v.at[c]], add=True)
        return 0
    lax.fori_loop(0, NCHUNK, chunk, 0)
    plsc.subcore_barrier()

    def ocopy(k, _):
        pltpu.sync_copy(acc.at[pl.ds(zbase + k * ZROWS, ZROWS)],
                        partial_hbm.at[cid, pl.ds(zbase + k * ZROWS, ZROWS)])
        return 0
    lax.fori_loop(0, nz, ocopy, 0)


@jax.jit
def _sc_scatter(x, src2, dst2, ew2, wmp_vec):
    mesh = plsc.VectorSubcoreMesh(
        core_axis_name="c", subcore_axis_name="s", num_cores=NC,
        num_subcores=NS)
    return pl.kernel(
        _sc_body,
        out_type=jax.ShapeDtypeStruct((NC, N, D), jnp.float32),
        mesh=mesh,
        scratch_types=[
            pltpu.VMEM((NCHUNK, C), jnp.int32),
            pltpu.VMEM((NCHUNK, C), jnp.int32),
            pltpu.VMEM((NCHUNK, C), jnp.float32),
            pltpu.VMEM((C, D), jnp.float32),
            pltpu.VMEM((L,), jnp.float32),
            pltpu.VMEM_SHARED((N, D), jnp.float32),
            pltpu.SemaphoreType.DMA,
        ],
    )(x, src2, dst2, ew2, wmp_vec)


def _tc_add_body(p_ref, o_ref):
    o_ref[...] = p_ref[0] + p_ref[1]


@jax.jit
def _tc_add(partial):
    blk = 1000
    return pl.pallas_call(
        _tc_add_body,
        grid=(N // blk,),
        in_specs=[pl.BlockSpec((NC, blk, D), lambda i: (0, i, 0))],
        out_specs=pl.BlockSpec((blk, D), lambda i: (i, 0)),
        out_shape=jax.ShapeDtypeStruct((N, D), jnp.float32),
    )(partial)


def kernel(x, edge_index, edge_weight, halo_info, mask_send, mask_recv,
           buffer_send, buffer_recv, neighboring_procs, SIZE, w_mp):
    src2 = edge_index[0].reshape(E // C, C)
    dst2 = edge_index[1].reshape(E // C, C)
    ew2 = edge_weight.reshape(E // C, C)
    wmp_vec = jnp.broadcast_to(w_mp.astype(jnp.float32), (L,))
    partial = _sc_scatter(x, src2, dst2, ew2, wmp_vec)
    return _tc_add(partial)


# speedup vs baseline: 6.8219x; 6.8219x over previous
import jax
import jax.numpy as jnp
from jax import lax
from jax.experimental import pallas as pl
from jax.experimental.pallas import tpu as pltpu
from jax.experimental.pallas import tpu_sc as plsc

N = 10000
E = 320000
D = 128
L = 16
NC = 2
NS = 16
NW = NC * NS
EW = E // NW
C = 80
NCHUNK = EW // C
G = C // L

ZROWS = 80
FULL_ZCHUNKS = 8
LAST_ZCHUNKS = 5


def _lane_bcast(vec, lane):
    idx = jnp.full((L, 1), lane, jnp.int32)
    dn = lax.GatherDimensionNumbers(
        offset_dims=(), collapsed_slice_dims=(0,), start_index_map=(0,))
    return lax.gather(vec, idx, dn, (1,),
                      mode=lax.GatherScatterMode.PROMISE_IN_BOUNDS)


def _sc_body(x_hbm, src_hbm, dst_hbm, ew_hbm, wmp_hbm, partial_hbm,
             src_v, dst_v, w_v, rows_v, wmp_v, acc, sem):
    cid = lax.axis_index("c")
    sid = lax.axis_index("s")
    wid = cid * NS + sid

    zero = jnp.zeros((L,), jnp.float32)
    def zfill(i, _):
        for j in range(D // L):
            rows_v[i, pl.ds(j * L, L)] = zero
        return 0
    lax.fori_loop(0, C, zfill, 0)
    nz = jnp.where(sid == NS - 1, LAST_ZCHUNKS, FULL_ZCHUNKS)
    zbase = sid * (FULL_ZCHUNKS * ZROWS)
    def zcopy(k, _):
        pltpu.sync_copy(rows_v, acc.at[pl.ds(zbase + k * ZROWS, ZROWS)])
        return 0
    lax.fori_loop(0, nz, zcopy, 0)
    plsc.subcore_barrier()

    pltpu.sync_copy(src_hbm.at[pl.ds(wid * EW, EW)], src_v)
    pltpu.sync_copy(dst_hbm.at[wid], dst_v)
    pltpu.sync_copy(ew_hbm.at[pl.ds(wid * EW, EW)], w_v)
    pltpu.sync_copy(wmp_hbm, wmp_v)
    wmp = wmp_v[...]

    def chunk(c, _):
        pltpu.async_copy(x_hbm.at[src_v.at[pl.ds(c * C, C)]], rows_v,
                         sem).wait()
        for g in range(G):
            wvec = w_v[pl.ds(c * C + g * L, L)] * wmp
            def row(r, _):
                i = g * L + r
                wb = _lane_bcast(wvec, r)
                for j in range(D // L):
                    rows_v[i, pl.ds(j * L, L)] = rows_v[i, pl.ds(j * L, L)] * wb
                return 0
            lax.fori_loop(0, L, row, 0)
        pltpu.sync_copy(rows_v, acc.at[dst_v.at[c]], add=True)
        return 0
    lax.fori_loop(0, NCHUNK, chunk, 0)
    plsc.subcore_barrier()

    def ocopy(k, _):
        pltpu.sync_copy(acc.at[pl.ds(zbase + k * ZROWS, ZROWS)],
                        partial_hbm.at[cid, pl.ds(zbase + k * ZROWS, ZROWS)])
        return 0
    lax.fori_loop(0, nz, ocopy, 0)


@jax.jit
def _sc_scatter(x, src3, dst3, ew3, wmp_vec):
    mesh = plsc.VectorSubcoreMesh(
        core_axis_name="c", subcore_axis_name="s", num_cores=NC,
        num_subcores=NS)
    return pl.kernel(
        _sc_body,
        out_type=jax.ShapeDtypeStruct((NC, N, D), jnp.float32),
        mesh=mesh,
        scratch_types=[
            pltpu.VMEM((EW,), jnp.int32),
            pltpu.VMEM((NCHUNK, C), jnp.int32),
            pltpu.VMEM((EW,), jnp.float32),
            pltpu.VMEM((C, D), jnp.float32),
            pltpu.VMEM((L,), jnp.float32),
            pltpu.VMEM_SHARED((N, D), jnp.float32),
            pltpu.SemaphoreType.DMA,
        ],
    )(x, src3, dst3, ew3, wmp_vec)


def _tc_add_body(p_ref, o_ref):
    o_ref[...] = p_ref[0] + p_ref[1]


@jax.jit
def _tc_add(partial):
    blk = 1000
    return pl.pallas_call(
        _tc_add_body,
        grid=(N // blk,),
        in_specs=[pl.BlockSpec((NC, blk, D), lambda i: (0, i, 0))],
        out_specs=pl.BlockSpec((blk, D), lambda i: (i, 0)),
        out_shape=jax.ShapeDtypeStruct((N, D), jnp.float32),
    )(partial)


def kernel(x, edge_index, edge_weight, halo_info, mask_send, mask_recv,
           buffer_send, buffer_recv, neighboring_procs, SIZE, w_mp):
    src3 = edge_index[0]
    dst3 = edge_index[1].reshape(NW, NCHUNK, C)
    ew3 = edge_weight
    wmp_vec = jnp.broadcast_to(w_mp.astype(jnp.float32), (L,))
    partial = _sc_scatter(x, src3, dst3, ew3, wmp_vec)
    return _tc_add(partial)
